# Initial kernel scaffold; baseline (speedup 1.0000x reference)
import jax
import jax.numpy as jnp
from jax.experimental import pallas as pl


def kernel(x, edge_index, batch, W1, b1, W2, b2, bn_g, bn_b, gin_eps, M1, mb1, M2, mb2, P1, pb1, pg, pbt, P2, pb2):
    raise NotImplementedError("write your pallas kernel here")


# speedup vs baseline: 2.7145x; 2.7145x over previous
import functools

import jax
import jax.numpy as jnp
from jax import lax
from jax.experimental import pallas as pl
from jax.experimental.pallas import tpu as pltpu
from jax.experimental.pallas import tpu_sc as plsc

_L = 5
_EMB = 128
_N = 10000
_E = 320000
_NG = 128

_NC = 2
_NS = 16
_NW = _NC * _NS
_K = 80
_NPAD = 10240
_RPT = _NPAD // _NS
_EPC = _E // _NC
_SZ_BIG, _SZ_MID, _SZ_END = 10080, 9840, 9760
_OFF_MID = 11 * _SZ_BIG
_OFF_END = _OFF_MID + 4 * _SZ_MID

_RB = 1000
_NB = _N // _RB


def _edge_agg_sc(r, src, mseg, fidx, zeros_rpt):
    mesh = plsc.VectorSubcoreMesh(
        core_axis_name="c", subcore_axis_name="s",
        num_cores=_NC, num_subcores=_NS)

    @functools.partial(
        pl.kernel,
        out_type=jax.ShapeDtypeStruct((_NC * _NPAD, _EMB), jnp.float32),
        mesh=mesh,
        scratch_types=[
            pltpu.VMEM((_K,), jnp.int32),
            pltpu.VMEM((_K,), jnp.int32),
            pltpu.VMEM((_K * 16,), jnp.float32),
            pltpu.VMEM((_K, _EMB), jnp.float32),
            pltpu.VMEM_SHARED((_NPAD, _EMB), jnp.float32),
            pltpu.SemaphoreType.DMA,
        ],
    )
    def body(r_hbm, src_hbm, m_hbm, f_hbm, zero_hbm, out_hbm, sidx, fvec,
             mvec, rows, acc, sem):
        cid = lax.axis_index("c")
        sid = lax.axis_index("s")
        pltpu.sync_copy(zero_hbm, acc.at[pl.ds(sid * _RPT, _RPT)])
        off = jnp.where(
            sid <= 10, sid * _SZ_BIG,
            jnp.where(sid <= 14, _OFF_MID + (sid - 11) * _SZ_MID, _OFF_END))
        ebase = cid * _EPC + off
        nchunk = jnp.where(sid <= 10, _SZ_BIG // _K,
                           jnp.where(sid <= 14, _SZ_MID // _K, _SZ_END // _K))
        plsc.subcore_barrier()
        zero16 = jnp.zeros((16,), jnp.float32)

        def chunk(i, carry):
            base = ebase + i * _K
            pltpu.sync_copy(src_hbm.at[pl.ds(base, _K)], sidx)
            pltpu.sync_copy(m_hbm.at[pl.ds(base * 16, _K * 16)], mvec)
            pltpu.sync_copy(f_hbm.at[pl.ds(base, _K)], fvec)
            pltpu.async_copy(r_hbm.at[sidx], rows, sem).wait()
            prev = list(carry)
            for e in range(_K):
                mb = mvec[pl.ds(e * 16, 16)]
                for k in range(8):
                    cur = rows[e, pl.ds(k * 16, 16)] + mb * prev[k]
                    rows[e, pl.ds(k * 16, 16)] = cur
                    prev[k] = cur
            pltpu.sync_copy(rows, acc.at[fvec], add=True)
            return tuple(prev)

        lax.fori_loop(0, nchunk, chunk, tuple(zero16 for _ in range(8)))
        plsc.subcore_barrier()
        pltpu.sync_copy(acc.at[pl.ds(sid * _RPT, _RPT)],
                        out_hbm.at[pl.ds(cid * _NPAD + sid * _RPT, _RPT)])

    return body(r, src, mseg, fidx, zeros_rpt)


def _relu_body(x_ref, o_ref):
    o_ref[...] = jnp.maximum(x_ref[...], 0.0)


def _relu_tc(x):
    return pl.pallas_call(
        _relu_body,
        grid=(_NB,),
        in_specs=[pl.BlockSpec((_RB, _EMB), lambda b: (b, 0))],
        out_specs=pl.BlockSpec((_RB, _EMB), lambda b: (b, 0)),
        out_shape=jax.ShapeDtypeStruct((_N, _EMB), jnp.float32),
    )(x)


def _dense_body(h_ref, p_ref, w1_ref, b1_ref, w2_ref, b2_ref, g_ref, bb_ref,
                eps_ref, hn_ref, rn_ref, z2s, ssum, svar, *, last):
    ph = pl.program_id(0)
    b = pl.program_id(1)

    @pl.when(ph == 0)
    def _phase0():
        agg = p_ref[0, :, :] + p_ref[1, :, :]
        z0 = eps_ref[...] * h_ref[...] + agg
        z1 = jnp.dot(z0, w1_ref[...],
                     preferred_element_type=jnp.float32) + b1_ref[...]
        z1 = jnp.maximum(z1, 0.0)
        z2 = jnp.dot(z1, w2_ref[...],
                     preferred_element_type=jnp.float32) + b2_ref[...]
        z2s[pl.ds(b * _RB, _RB), :] = z2

        @pl.when(b == 0)
        def _zero():
            ssum[...] = jnp.zeros_like(ssum)

        ssum[...] += jnp.sum(z2, axis=0, keepdims=True)

    @pl.when(ph == 1)
    def _phase1():
        mu = ssum[...] * (1.0 / _N)
        z2 = z2s[pl.ds(b * _RB, _RB), :]

        @pl.when(b == 0)
        def _zero():
            svar[...] = jnp.zeros_like(svar)

        dd = z2 - mu
        svar[...] += jnp.sum(dd * dd, axis=0, keepdims=True)

    @pl.when(ph == 2)
    def _phase2():
        mu = ssum[...] * (1.0 / _N)
        var = svar[...] * (1.0 / _N)
        z2 = z2s[pl.ds(b * _RB, _RB), :]
        zn = (z2 - mu) / jnp.sqrt(var + 1e-5) * g_ref[...] + bb_ref[...]
        if not last:
            zn = jnp.maximum(zn, 0.0)
        hn = zn + h_ref[...]
        hn_ref[...] = hn
        if not last:
            rn_ref[...] = jnp.maximum(hn, 0.0)


def _dense_layer_tc(h, parts, w1, b1, w2, b2, g, bb, epsv, last):
    full = lambda p, b: (0, 0)
    out_map = lambda p, b: ((p // 2) * b, 0)
    rn_map = (lambda p, b: (0, 0)) if last else out_map
    return pl.pallas_call(
        functools.partial(_dense_body, last=last),
        grid=(3, _NB),
        in_specs=[
            pl.BlockSpec((_RB, _EMB), lambda p, b: (b, 0)),
            pl.BlockSpec((2, _RB, _EMB), lambda p, b: (0, b, 0)),
            pl.BlockSpec((_EMB, 2 * _EMB), full),
            pl.BlockSpec((1, 2 * _EMB), full),
            pl.BlockSpec((2 * _EMB, _EMB), full),
            pl.BlockSpec((1, _EMB), full),
            pl.BlockSpec((1, _EMB), full),
            pl.BlockSpec((1, _EMB), full),
            pl.BlockSpec((1, _EMB), full),
        ],
        out_specs=[
            pl.BlockSpec((_RB, _EMB), out_map),
            pl.BlockSpec((_RB, _EMB), rn_map),
        ],
        out_shape=[
            jax.ShapeDtypeStruct((_N, _EMB), jnp.float32),
            jax.ShapeDtypeStruct((_N, _EMB), jnp.float32),
        ],
        scratch_shapes=[
            pltpu.VMEM((_N, _EMB), jnp.float32),
            pltpu.VMEM((1, _EMB), jnp.float32),
            pltpu.VMEM((1, _EMB), jnp.float32),
        ],
    )(h, parts, w1, b1, w2, b2, g, bb, epsv)


def _pool_body(h_ref, bt_ref, m1_ref, mb1_ref, m2_ref, mb2_ref, p1_ref,
               pb1_ref, pg_ref, pbt_ref, p2_ref, pb2_ref, out_ref,
               hsum, cmat):
    b = pl.program_id(0)

    @pl.when(b == 0)
    def _zero():
        hsum[...] = jnp.zeros_like(hsum)
        cmat[...] = jnp.zeros_like(cmat)

    bt = bt_ref[0, :, :]
    oh = (lax.broadcasted_iota(jnp.int32, (_NG, _RB), 0) == bt).astype(
        jnp.float32)
    hsum[...] += jnp.dot(oh, h_ref[...], preferred_element_type=jnp.float32,
                         precision=lax.Precision.HIGHEST)
    cmat[...] += jnp.dot(oh, jnp.ones((_RB, _NG), jnp.float32),
                         preferred_element_type=jnp.float32,
                         precision=lax.Precision.HIGHEST)

    @pl.when(b == _NB - 1)
    def _final():
        hs = hsum[...]
        cm = jnp.maximum(cmat[...], 1.0)
        hm = hs / cm
        a = jnp.dot(hm, m1_ref[...],
                    preferred_element_type=jnp.float32) + mb1_ref[...]
        a = jnp.maximum(a, 0.0)
        alpha = jax.nn.sigmoid(
            jnp.dot(a, m2_ref[...],
                    preferred_element_type=jnp.float32) + mb2_ref[...])
        hp = alpha * hs + (1.0 - alpha) * hm
        p = jnp.dot(hp, p1_ref[...],
                    preferred_element_type=jnp.float32) + pb1_ref[...]
        mu = jnp.sum(p, axis=0, keepdims=True) * (1.0 / _NG)
        dd = p - mu
        var = jnp.sum(dd * dd, axis=0, keepdims=True) * (1.0 / _NG)
        p = (p - mu) / jnp.sqrt(var + 1e-5) * pg_ref[...] + pbt_ref[...]
        p = jnp.maximum(p, 0.0)
        out_ref[...] = jnp.dot(p, p2_ref[...],
                               preferred_element_type=jnp.float32) + pb2_ref[...]


def _pool_tc(h, batch3, m1, mb1, m2t, mb2t, p1, pb1, pg, pbt, p2t, pb2t):
    full = lambda b: (0, 0)
    return pl.pallas_call(
        _pool_body,
        grid=(_NB,),
        in_specs=[
            pl.BlockSpec((_RB, _EMB), lambda b: (b, 0)),
            pl.BlockSpec((1, 1, _RB), lambda b: (b, 0, 0)),
            pl.BlockSpec((_EMB, _EMB), full),
            pl.BlockSpec((1, _EMB), full),
            pl.BlockSpec((_EMB, _NG), full),
            pl.BlockSpec((1, _NG), full),
            pl.BlockSpec((_EMB, 2 * _EMB), full),
            pl.BlockSpec((1, 2 * _EMB), full),
            pl.BlockSpec((1, 2 * _EMB), full),
            pl.BlockSpec((1, 2 * _EMB), full),
            pl.BlockSpec((2 * _EMB, _NG), full),
            pl.BlockSpec((1, _NG), full),
        ],
        out_specs=pl.BlockSpec((_NG, _NG), full),
        out_shape=jax.ShapeDtypeStruct((_NG, _NG), jnp.float32),
        scratch_shapes=[
            pltpu.VMEM((_NG, _NG), jnp.float32),
            pltpu.VMEM((_NG, _NG), jnp.float32),
        ],
    )(h, batch3, m1, mb1, m2t, mb2t, p1, pb1, pg, pbt, p2t, pb2t)


def kernel(x, edge_index, batch, W1, b1, W2, b2, bn_g, bn_b, gin_eps,
           M1, mb1, M2, mb2, P1, pb1, pg, pbt, P2, pb2):
    src = edge_index[0]
    dst = edge_index[1]
    sdst, ssrc = lax.sort_key_val(dst, src)
    starts = []
    for c in range(_NC):
        for s in range(_NS):
            if s <= 10:
                o = s * _SZ_BIG
            elif s <= 14:
                o = _OFF_MID + (s - 11) * _SZ_MID
            else:
                o = _OFF_END
            starts.append(c * _EPC + o)
    starts_arr = jnp.asarray(starts, jnp.int32)
    same_prev = jnp.concatenate(
        [jnp.zeros((1,), bool), sdst[1:] == sdst[:-1]])
    range_start = jnp.zeros((_E,), bool).at[starts_arr].set(True)
    mseg = jnp.repeat(
        jnp.where(same_prev & ~range_start, 1.0, 0.0).astype(jnp.float32), 16)
    seg_end = jnp.concatenate([sdst[:-1] != sdst[1:], jnp.ones((1,), bool)])
    range_end = jnp.zeros((_E,), bool).at[starts_arr[1:] - 1].set(True)
    fidx = jnp.where(seg_end | range_end, sdst, _N).astype(jnp.int32)
    zeros_rpt = jnp.zeros((_RPT, _EMB), jnp.float32)

    h = x
    r = _relu_tc(x)
    for l in range(_L):
        parts = _edge_agg_sc(r, ssrc, mseg, fidx, zeros_rpt).reshape(
            _NC, _NPAD, _EMB)
        epsv = jnp.full((1, _EMB), 1.0, jnp.float32) + gin_eps[l]
        h, r = _dense_layer_tc(
            h, parts, W1[l], b1[l].reshape(1, -1), W2[l],
            b2[l].reshape(1, -1), bn_g[l].reshape(1, -1),
            bn_b[l].reshape(1, -1), epsv, last=(l == _L - 1))

    nt = P2.shape[1]
    batch3 = batch.reshape(_NB, 1, _RB)
    m2t = jnp.tile(M2, (1, _NG // M2.shape[1]))
    mb2t = jnp.tile(mb2.reshape(1, -1), (1, _NG // mb2.shape[0]))
    p2t = jnp.tile(P2, (1, _NG // nt))
    pb2t = jnp.tile(pb2.reshape(1, -1), (1, _NG // nt))
    out = _pool_tc(h, batch3, M1, mb1.reshape(1, -1), m2t, mb2t, P1,
                   pb1.reshape(1, -1), pg.reshape(1, -1), pbt.reshape(1, -1),
                   p2t, pb2t)
    pred = out[:, :nt]
    return pred, jnp.float32(0.0)
